# baseline (speedup 1.0000x reference)
import functools
import jax
import jax.numpy as jnp
from jax import lax
from jax.experimental import pallas as pl
from jax.experimental.pallas import tpu as pltpu

N = 262144
C = 170
_BR = 2048
_NB = N // _BR


def _tc_body(tgt_ref, logits_ref, out_ref):
    i = pl.program_id(0)

    @pl.when(i == 0)
    def _():
        out_ref[0, 0] = 0.0

    x = logits_ref[...]
    t = tgt_ref[0, 0, :]
    cols = lax.broadcasted_iota(jnp.int32, (_BR, C), 1)
    sel = jnp.where(cols == t[:, None], (1.0 - x) * (1.0 - x), 0.0)
    out_ref[0, 0] += jnp.sum(sel)


@jax.jit
def kernel(contrast_logits, contrast_target):
    tgt = contrast_target.astype(jnp.int32).reshape(_NB, 1, _BR)
    total = pl.pallas_call(
        _tc_body,
        grid=(_NB,),
        in_specs=[
            pl.BlockSpec((1, 1, _BR), lambda i: (i, 0, 0)),
            pl.BlockSpec((_BR, C), lambda i: (i, 0)),
        ],
        out_specs=pl.BlockSpec((1, 1), lambda i: (0, 0), memory_space=pltpu.SMEM),
        out_shape=jax.ShapeDtypeStruct((1, 1), jnp.float32),
        compiler_params=pltpu.CompilerParams(
            dimension_semantics=("arbitrary",),
        ),
    )(tgt, contrast_logits)
    return total[0, 0] / N
